# initial kernel scaffold (unmeasured)
import jax
import jax.numpy as jnp
from jax import lax
from jax.experimental import pallas as pl
from jax.experimental.pallas import tpu as pltpu

SCALE = 64 ** -0.5


def _partial_body(q_ref, k_ref, v_ref, onum_ref, m_ref, l_ref):
    q = q_ref[0] * SCALE
    k = k_ref[0]
    v = v_ref[0]
    s = jnp.sum(k * q, axis=2, keepdims=True)
    m = jnp.max(s, axis=0, keepdims=True)
    p = jnp.exp(s - m)
    l = jnp.sum(p, axis=0, keepdims=True)
    onum = jnp.sum(p * v, axis=0)
    onum_ref[0, 0] = onum
    m_ref[...] = m
    l_ref[...] = l


def _merge_body(onum_ref, stats_ref, out_ref,
                r_onum, r_stats, send_sems, recv_sems):
    my_x = lax.axis_index("x")
    my_y = lax.axis_index("y")
    my_z = lax.axis_index("z")
    partner = (my_x, my_y, 1 - my_z)

    copies = []
    for i, (src, dst) in enumerate(((onum_ref, r_onum), (stats_ref, r_stats))):
        rdma = pltpu.make_async_remote_copy(
            src_ref=src,
            dst_ref=dst,
            send_sem=send_sems.at[i],
            recv_sem=recv_sems.at[i],
            device_id=partner,
            device_id_type=pl.DeviceIdType.MESH,
        )
        rdma.start()
        copies.append(rdma)
    for rdma in copies:
        rdma.wait()

    m0 = stats_ref[:, 0:1]
    l0 = stats_ref[:, 1:2]
    m1 = r_stats[:, 0:1]
    l1 = r_stats[:, 1:2]
    mg = jnp.maximum(m0, m1)
    a0 = jnp.exp(m0 - mg)
    a1 = jnp.exp(m1 - mg)
    lg = a0 * l0 + a1 * l1
    out_ref[...] = (onum_ref[...] * a0 + r_onum[...] * a1) / lg


def kernel(Q, K, V):
    b, sq, h, d = Q.shape
    skv = K.shape[1]

    onum, m, l = pl.pallas_call(
        _partial_body,
        grid=(b,),
        in_specs=[
            pl.BlockSpec((1, sq, h, d), lambda i: (i, 0, 0, 0)),
            pl.BlockSpec((1, skv, h, d), lambda i: (i, 0, 0, 0)),
            pl.BlockSpec((1, skv, h, d), lambda i: (i, 0, 0, 0)),
        ],
        out_specs=[
            pl.BlockSpec((1, sq, h, d), lambda i: (i, 0, 0, 0)),
            pl.BlockSpec((1, h, 1), lambda i: (i, 0, 0)),
            pl.BlockSpec((1, h, 1), lambda i: (i, 0, 0)),
        ],
        out_shape=[
            jax.ShapeDtypeStruct((b, sq, h, d), jnp.float32),
            jax.ShapeDtypeStruct((b, h, 1), jnp.float32),
            jax.ShapeDtypeStruct((b, h, 1), jnp.float32),
        ],
    )(Q, K, V)

    stats = jnp.stack([m, l], axis=1)

    out = pl.pallas_call(
        _merge_body,
        in_specs=[
            pl.BlockSpec(memory_space=pltpu.VMEM),
            pl.BlockSpec(memory_space=pltpu.VMEM),
        ],
        out_specs=pl.BlockSpec(memory_space=pltpu.VMEM),
        out_shape=jax.ShapeDtypeStruct((b, sq, h, d), jnp.float32),
        scratch_shapes=[
            pltpu.VMEM((b, sq, h, d), jnp.float32),
            pltpu.VMEM((b, 2, h, 1), jnp.float32),
            pltpu.SemaphoreType.DMA((2,)),
            pltpu.SemaphoreType.DMA((2,)),
        ],
        compiler_params=pltpu.CompilerParams(has_side_effects=True),
    )(onum, stats)
    return out


# baseline (device time: 325882 ns/iter reference)
import jax
import jax.numpy as jnp
from jax import lax
from jax.experimental import pallas as pl
from jax.experimental.pallas import tpu as pltpu

SCALE = 64 ** -0.5


def _partial_body(q_ref, k_ref, v_ref, onum_ref, m_ref, l_ref):
    q = q_ref[0] * SCALE
    k = k_ref[0]
    v = v_ref[0]
    s = jnp.sum(k * q, axis=2, keepdims=True)
    m = jnp.max(s, axis=0, keepdims=True)
    p = jnp.exp(s - m)
    l = jnp.sum(p, axis=0, keepdims=True)
    onum = jnp.sum(p * v, axis=0)
    onum_ref[0, 0] = onum
    m_ref[...] = m
    l_ref[...] = l


def _merge_body(onum_ref, stats_ref, out_ref,
                r_onum, r_stats, send_sems, recv_sems):
    my_x = lax.axis_index("x")
    my_y = lax.axis_index("y")
    my_z = lax.axis_index("z")
    partner = (my_x, my_y, 1 - my_z)

    copies = []
    for i, (src, dst) in enumerate(((onum_ref, r_onum), (stats_ref, r_stats))):
        rdma = pltpu.make_async_remote_copy(
            src_ref=src,
            dst_ref=dst,
            send_sem=send_sems.at[i],
            recv_sem=recv_sems.at[i],
            device_id=partner,
            device_id_type=pl.DeviceIdType.MESH,
        )
        rdma.start()
        copies.append(rdma)
    for rdma in copies:
        rdma.wait()

    m0 = stats_ref[:, 0:1]
    l0 = stats_ref[:, 1:2]
    m1 = r_stats[:, 0:1]
    l1 = r_stats[:, 1:2]
    mg = jnp.maximum(m0, m1)
    a0 = jnp.exp(m0 - mg)
    a1 = jnp.exp(m1 - mg)
    lg = a0 * l0 + a1 * l1
    out_ref[...] = (onum_ref[...] * a0 + r_onum[...] * a1) / lg


def kernel(Q, K, V):
    b, sq, h, d = Q.shape
    skv = K.shape[1]

    onum, m, l = pl.pallas_call(
        _partial_body,
        grid=(b,),
        in_specs=[
            pl.BlockSpec((1, sq, h, d), lambda i: (i, 0, 0, 0)),
            pl.BlockSpec((1, skv, h, d), lambda i: (i, 0, 0, 0)),
            pl.BlockSpec((1, skv, h, d), lambda i: (i, 0, 0, 0)),
        ],
        out_specs=[
            pl.BlockSpec((1, sq, h, d), lambda i: (i, 0, 0, 0)),
            pl.BlockSpec((1, h, 1), lambda i: (i, 0, 0)),
            pl.BlockSpec((1, h, 1), lambda i: (i, 0, 0)),
        ],
        out_shape=[
            jax.ShapeDtypeStruct((b, sq, h, d), jnp.float32),
            jax.ShapeDtypeStruct((b, h, 1), jnp.float32),
            jax.ShapeDtypeStruct((b, h, 1), jnp.float32),
        ],
        compiler_params=pltpu.CompilerParams(
            vmem_limit_bytes=96 * 1024 * 1024,
        ),
    )(Q, K, V)

    stats = jnp.stack([m, l], axis=1)

    out = pl.pallas_call(
        _merge_body,
        in_specs=[
            pl.BlockSpec(memory_space=pltpu.VMEM),
            pl.BlockSpec(memory_space=pltpu.VMEM),
        ],
        out_specs=pl.BlockSpec(memory_space=pltpu.VMEM),
        out_shape=jax.ShapeDtypeStruct((b, sq, h, d), jnp.float32),
        scratch_shapes=[
            pltpu.VMEM((b, sq, h, d), jnp.float32),
            pltpu.VMEM((b, 2, h, 1), jnp.float32),
            pltpu.SemaphoreType.DMA((2,)),
            pltpu.SemaphoreType.DMA((2,)),
        ],
        compiler_params=pltpu.CompilerParams(has_side_effects=True),
    )(onum, stats)
    return out


# device time: 316446 ns/iter; 1.0298x vs baseline; 1.0298x over previous
import jax
import jax.numpy as jnp
from jax import lax
from jax.experimental import pallas as pl
from jax.experimental.pallas import tpu as pltpu

SCALE = 64 ** -0.5


def _partial_body(q_ref, k_ref, v_ref, mask_ref, onum_ref, m_ref, l_ref):
    q = q_ref[0] * SCALE
    k = k_ref[0]
    v = v_ref[0]
    s = lax.dot_general(
        q, k, (((1,), (1,)), ((), ())),
        preferred_element_type=jnp.float32,
    )
    m = jnp.max(s, axis=1, keepdims=True)
    p = jnp.exp(s - m) * mask_ref[...]
    l = jnp.sum(p, axis=1, keepdims=True)
    o = lax.dot_general(
        p, v, (((1,), (0,)), ((), ())),
        preferred_element_type=jnp.float32,
    )
    onum_ref[0] = o
    m_ref[0] = m
    l_ref[0] = l


def _merge_body(onum_ref, stats_ref, out_ref,
                r_onum, r_stats, send_sems, recv_sems):
    my_x = lax.axis_index("x")
    my_y = lax.axis_index("y")
    my_z = lax.axis_index("z")
    partner = (my_x, my_y, 1 - my_z)

    copies = []
    for i, (src, dst) in enumerate(((onum_ref, r_onum), (stats_ref, r_stats))):
        rdma = pltpu.make_async_remote_copy(
            src_ref=src,
            dst_ref=dst,
            send_sem=send_sems.at[i],
            recv_sem=recv_sems.at[i],
            device_id=partner,
            device_id_type=pl.DeviceIdType.MESH,
        )
        rdma.start()
        copies.append(rdma)
    for rdma in copies:
        rdma.wait()

    m0 = stats_ref[:, 0]
    l0 = stats_ref[:, 1]
    m1 = r_stats[:, 0]
    l1 = r_stats[:, 1]
    mg = jnp.maximum(m0, m1)
    a0 = jnp.exp(m0 - mg)
    a1 = jnp.exp(m1 - mg)
    lg = a0 * l0 + a1 * l1
    out_ref[...] = (onum_ref[...] * a0 + r_onum[...] * a1) / lg


def kernel(Q, K, V):
    b, sq, h, d = Q.shape
    skv = K.shape[1]
    kh = skv * h

    q3 = Q.reshape(b, h, d)
    k3 = K.reshape(b, kh, d)
    v3 = V.reshape(b, kh, d)

    cols = jax.lax.broadcasted_iota(jnp.int32, (h, kh), 1)
    rows = jax.lax.broadcasted_iota(jnp.int32, (h, kh), 0)
    mask = (cols % h == rows).astype(jnp.float32)

    onum, m, l = pl.pallas_call(
        _partial_body,
        grid=(b,),
        in_specs=[
            pl.BlockSpec((1, h, d), lambda i: (i, 0, 0)),
            pl.BlockSpec((1, kh, d), lambda i: (i, 0, 0)),
            pl.BlockSpec((1, kh, d), lambda i: (i, 0, 0)),
            pl.BlockSpec((h, kh), lambda i: (0, 0)),
        ],
        out_specs=[
            pl.BlockSpec((1, h, d), lambda i: (i, 0, 0)),
            pl.BlockSpec((1, h, 1), lambda i: (i, 0, 0)),
            pl.BlockSpec((1, h, 1), lambda i: (i, 0, 0)),
        ],
        out_shape=[
            jax.ShapeDtypeStruct((b, h, d), jnp.float32),
            jax.ShapeDtypeStruct((b, h, 1), jnp.float32),
            jax.ShapeDtypeStruct((b, h, 1), jnp.float32),
        ],
        compiler_params=pltpu.CompilerParams(
            vmem_limit_bytes=96 * 1024 * 1024,
        ),
    )(q3, k3, v3, mask)

    stats = jnp.stack([m, l], axis=1)

    out = pl.pallas_call(
        _merge_body,
        in_specs=[
            pl.BlockSpec(memory_space=pltpu.VMEM),
            pl.BlockSpec(memory_space=pltpu.VMEM),
        ],
        out_specs=pl.BlockSpec(memory_space=pltpu.VMEM),
        out_shape=jax.ShapeDtypeStruct((b, h, d), jnp.float32),
        scratch_shapes=[
            pltpu.VMEM((b, h, d), jnp.float32),
            pltpu.VMEM((b, 2, h, 1), jnp.float32),
            pltpu.SemaphoreType.DMA((2,)),
            pltpu.SemaphoreType.DMA((2,)),
        ],
        compiler_params=pltpu.CompilerParams(has_side_effects=True),
    )(onum, stats)
    return out.reshape(b, sq, h, d)
